# Initial kernel scaffold; baseline (speedup 1.0000x reference)
import jax
import jax.numpy as jnp
from jax.experimental import pallas as pl


def kernel(edge_index, edge_weights, num_nodes):
    raise NotImplementedError("write your pallas kernel here")


# speedup vs baseline: 1.8120x; 1.8120x over previous
import functools

import jax
import jax.numpy as jnp
from jax import lax
from jax.experimental import pallas as pl
from jax.experimental.pallas import tpu as pltpu
from jax.experimental.pallas import tpu_sc as plsc

N_NODES = 50000
B_STATIC = 4
E_STATIC = 1600000
VARPHI = 0.7
NUM_CORES = 2
NUM_SUBCORES = 16
NW = NUM_CORES * NUM_SUBCORES
EDGES_PER_TILE = E_STATIC // NW
CH = 2000
NCHUNK = EDGES_PER_TILE // CH
GROUPS = (2 * CH) // 16
K_MEDIAN = (N_NODES - 1) // 2


def _sc_degree_partials(eidx_flat, weights):
    mesh = plsc.VectorSubcoreMesh(core_axis_name="c", subcore_axis_name="s")

    @functools.partial(
        pl.kernel,
        mesh=mesh,
        out_type=jax.ShapeDtypeStruct((B_STATIC * NW * N_NODES,), jnp.float32),
        scratch_types=[
            pltpu.VMEM((N_NODES,), jnp.float32),
            pltpu.VMEM((2 * CH,), jnp.int32),
            pltpu.VMEM((2 * CH,), jnp.int32),
            pltpu.VMEM((CH,), jnp.float32),
            pltpu.VMEM((CH,), jnp.float32),
            pltpu.SemaphoreType.DMA,
            pltpu.SemaphoreType.DMA,
            pltpu.SemaphoreType.DMA,
            pltpu.SemaphoreType.DMA,
        ],
        compiler_params=pltpu.CompilerParams(needs_layout_passes=False),
    )
    def sc_kernel(eidx_hbm, w_hbm, out_hbm, deg, eb0, eb1, wb0, wb1,
                  se0, se1, sw0, sw1):
        wid = lax.axis_index("s") * NUM_CORES + lax.axis_index("c")
        half = lax.shift_right_logical(lax.iota(jnp.int32, 16),
                                       jnp.int32(1))
        ebufs = (eb0, eb1)
        wbufs = (wb0, wb1)
        esems = (se0, se1)
        wsems = (sw0, sw1)
        zeros16 = jnp.zeros((16,), jnp.float32)
        base = wid * EDGES_PER_TILE

        for b in range(B_STATIC):
            def zero_body(i, carry):
                deg[pl.ds(i * 16, 16)] = zeros16
                return carry
            lax.fori_loop(0, N_NODES // 16, zero_body, 0)

            def start(c):
                lo = base + c * CH
                i = c % 2
                he = pltpu.async_copy(
                    eidx_hbm.at[pl.ds(b * 2 * E_STATIC + 2 * lo, 2 * CH)],
                    ebufs[i], esems[i])
                hw = pltpu.async_copy(
                    w_hbm.at[pl.ds(b * E_STATIC + lo, CH)],
                    wbufs[i], wsems[i])
                return he, hw

            pending = {0: start(0)}
            for c in range(NCHUNK):
                if c + 1 < NCHUNK:
                    pending[c + 1] = start(c + 1)
                he, hw = pending.pop(c)
                he.wait()
                hw.wait()
                eb = ebufs[c % 2]
                wb = wbufs[c % 2]

                def group_body(g, carry):
                    ev = eb[pl.ds(g * 16, 16)]
                    wv = plsc.load_gather(wb, [g * 8 + half])
                    plsc.addupdate_scatter(deg, [ev], wv)
                    return carry
                lax.fori_loop(0, GROUPS, group_body, 0)

            pltpu.sync_copy(
                deg, out_hbm.at[pl.ds((b * NW + wid) * N_NODES, N_NODES)])

    return sc_kernel(eidx_flat, weights)


def _tc_threshold(partials):
    def body(p_ref, o_ref):
        deg = p_ref[:, 0, :]
        for w in range(1, NW):
            deg = deg + p_ref[:, w, :]
        rmax = jnp.max(deg, axis=1, keepdims=True)
        xu = lax.bitcast_convert_type(deg, jnp.uint32)

        def radix_body(i, p):
            bit = (31 - i).astype(jnp.uint32)
            t = p | (jnp.uint32(1) << bit)
            cnt = jnp.sum((xu < t).astype(jnp.int32), axis=1, keepdims=True)
            return jnp.where(cnt <= K_MEDIAN, t, p)

        p = lax.fori_loop(0, 32, radix_body,
                          jnp.zeros((B_STATIC, 1), jnp.uint32))
        kth = lax.bitcast_convert_type(p, jnp.float32)
        denom = rmax + 1e-8
        norm = deg / denom
        thr = kth / denom
        o_ref[...] = jnp.where(norm >= thr, jnp.float32(1.0),
                               jnp.float32(VARPHI))

    return pl.pallas_call(
        body,
        out_shape=jax.ShapeDtypeStruct((B_STATIC, N_NODES), jnp.float32),
    )(partials)


def kernel(edge_index, edge_weights, num_nodes):
    eidx_flat = edge_index.reshape(-1)
    w_flat = edge_weights.reshape(-1)
    partials = _sc_degree_partials(eidx_flat, w_flat)
    partials = partials.reshape(B_STATIC, NW, N_NODES)
    return _tc_threshold(partials)


# speedup vs baseline: 49.5651x; 27.3543x over previous
import functools

import jax
import jax.numpy as jnp
from jax import lax
from jax.experimental import pallas as pl
from jax.experimental.pallas import tpu as pltpu
from jax.experimental.pallas import tpu_sc as plsc

N_NODES = 50000
N_PAD = 50048
B_STATIC = 4
E_STATIC = 1600000
VARPHI = 0.7
NUM_CORES = 2
NUM_SUBCORES = 16
NW = NUM_CORES * NUM_SUBCORES
GROUPS_PER_BATCH = 8
EDGES_PER_TILE = E_STATIC // GROUPS_PER_BATCH
CH = 2000
NCHUNK = EDGES_PER_TILE // CH
WCH = 2176
K_MEDIAN = (N_NODES - 1) // 2


def _sc_degree_partials(eidx_lin, weights):
    mesh = plsc.VectorSubcoreMesh(core_axis_name="c", subcore_axis_name="s")

    @functools.partial(
        pl.kernel,
        mesh=mesh,
        out_type=jax.ShapeDtypeStruct((NW * N_PAD,), jnp.float32),
        scratch_types=[
            pltpu.VMEM((N_PAD,), jnp.float32),
            pltpu.VMEM((CH,), jnp.int32),
            pltpu.VMEM((CH,), jnp.int32),
            pltpu.VMEM((CH,), jnp.int32),
            pltpu.VMEM((CH,), jnp.int32),
            pltpu.VMEM((B_STATIC, WCH), jnp.float32),
            pltpu.VMEM((B_STATIC, WCH), jnp.float32),
            pltpu.SemaphoreType.DMA,
            pltpu.SemaphoreType.DMA,
            pltpu.SemaphoreType.DMA,
            pltpu.SemaphoreType.DMA,
            pltpu.SemaphoreType.DMA,
            pltpu.SemaphoreType.DMA,
        ],
        compiler_params=pltpu.CompilerParams(needs_layout_passes=False),
    )
    def sc_kernel(eidx_hbm, w_hbm, out_hbm, deg, sb0, sb1, db0, db1,
                  wb0, wb1, ss0, ss1, sd0, sd1, sw0, sw1):
        wid = lax.axis_index("s") * NUM_CORES + lax.axis_index("c")
        b = lax.shift_right_logical(wid, 3)
        g = lax.bitwise_and(wid, 7)
        e_base = g * EDGES_PER_TILE
        src_base = b * (2 * E_STATIC) + e_base
        dst_base = src_base + E_STATIC
        sbufs = (sb0, sb1)
        dbufs = (db0, db1)
        wbufs = (wb0, wb1)
        ssems = (ss0, ss1)
        dsems = (sd0, sd1)
        wsems = (sw0, sw1)
        zeros16 = jnp.zeros((16,), jnp.float32)

        def zero_body(i, carry):
            deg[pl.ds(i * 16, 16)] = zeros16
            return carry
        lax.fori_loop(0, N_PAD // 16, zero_body, 0)

        def fire(c, i):
            off = c * CH
            e0 = e_base + off
            wa = pl.multiple_of(lax.bitwise_and(e0, ~127), 128)
            pltpu.async_copy(eidx_hbm.at[pl.ds(src_base + off, CH)],
                             sbufs[i], ssems[i])
            pltpu.async_copy(eidx_hbm.at[pl.ds(dst_base + off, CH)],
                             dbufs[i], dsems[i])
            pltpu.async_copy(w_hbm.at[:, pl.ds(wa, WCH)],
                             wbufs[i], wsems[i])

        def wait(i):
            pltpu.make_async_copy(eidx_hbm.at[pl.ds(0, CH)],
                                  sbufs[i], ssems[i]).wait()
            pltpu.make_async_copy(eidx_hbm.at[pl.ds(0, CH)],
                                  dbufs[i], dsems[i]).wait()
            pltpu.make_async_copy(w_hbm.at[:, pl.ds(0, WCH)],
                                  wbufs[i], wsems[i]).wait()

        def process(c, i):
            sb = sbufs[i]
            db = dbufs[i]
            wb = wbufs[i]
            woff = lax.bitwise_and(e_base + c * CH, 127)

            def grp_body(q, carry):
                o = q * 80
                for u in range(5):
                    ou = o + u * 16
                    sv = sb[pl.ds(ou, 16)]
                    dv = db[pl.ds(ou, 16)]
                    wv = wb[b, pl.ds(woff + ou, 16)]
                    plsc.addupdate_scatter(deg, [sv], wv)
                    plsc.addupdate_scatter(deg, [dv], wv)
                return carry
            lax.fori_loop(0, CH // 80, grp_body, 0)

        fire(0, 0)

        def pipe_body(t, carry):
            c0 = 2 * t
            c1 = 2 * t + 1
            fire(c1, 1)
            wait(0)
            process(c0, 0)

            @pl.when(c0 + 2 < NCHUNK)
            def _():
                fire(c0 + 2, 0)

            wait(1)
            process(c1, 1)
            return carry
        lax.fori_loop(0, NCHUNK // 2, pipe_body, 0)

        pltpu.sync_copy(deg, out_hbm.at[pl.ds(wid * N_PAD, N_PAD)])

    return sc_kernel(eidx_lin, weights)


def _tc_threshold(partials):
    def body(p_ref, o_ref):
        deg = p_ref[:, 0, :]
        for w in range(1, GROUPS_PER_BATCH):
            deg = deg + p_ref[:, w, :]
        rmax = jnp.max(deg, axis=1, keepdims=True)
        xu = lax.bitcast_convert_type(deg, jnp.uint32)
        col = lax.broadcasted_iota(jnp.int32, (B_STATIC, N_PAD), 1)
        valid = col < N_NODES

        def radix_body(i, p):
            bit = (31 - i).astype(jnp.uint32)
            t = p | (jnp.uint32(1) << bit)
            hit = jnp.logical_and(xu < t, valid)
            cnt = jnp.sum(hit.astype(jnp.int32), axis=1, keepdims=True)
            return jnp.where(cnt <= K_MEDIAN, t, p)

        p = lax.fori_loop(0, 32, radix_body,
                          jnp.zeros((B_STATIC, 1), jnp.uint32))
        kth = lax.bitcast_convert_type(p, jnp.float32)
        denom = rmax + 1e-8
        norm = deg / denom
        thr = kth / denom
        phi = jnp.where(norm >= thr, jnp.float32(1.0), jnp.float32(VARPHI))
        o_ref[...] = phi[:, :N_NODES]

    return pl.pallas_call(
        body,
        out_shape=jax.ShapeDtypeStruct((B_STATIC, N_NODES), jnp.float32),
    )(partials)


def kernel(edge_index, edge_weights, num_nodes):
    eidx_lin = edge_index.transpose(0, 2, 1).reshape(-1)
    partials = _sc_degree_partials(eidx_lin, edge_weights)
    partials = partials.reshape(B_STATIC, GROUPS_PER_BATCH, N_PAD)
    return _tc_threshold(partials)


# speedup vs baseline: 51.2275x; 1.0335x over previous
import functools

import jax
import jax.numpy as jnp
from jax import lax
from jax.experimental import pallas as pl
from jax.experimental.pallas import tpu as pltpu
from jax.experimental.pallas import tpu_sc as plsc

N_NODES = 50000
N_PAD = 50048
B_STATIC = 4
E_STATIC = 1600000
VARPHI = 0.7
NUM_CORES = 2
NUM_SUBCORES = 16
NW = NUM_CORES * NUM_SUBCORES
GROUPS_PER_BATCH = 8
EDGES_PER_TILE = E_STATIC // GROUPS_PER_BATCH
CH = 4000
NCHUNK = EDGES_PER_TILE // CH
WCH = 4224
K_MEDIAN = (N_NODES - 1) // 2


def _sc_degree_partials(eidx_lin, weights):
    mesh = plsc.VectorSubcoreMesh(core_axis_name="c", subcore_axis_name="s")

    @functools.partial(
        pl.kernel,
        mesh=mesh,
        out_type=jax.ShapeDtypeStruct((NW * N_PAD,), jnp.float32),
        scratch_types=[
            pltpu.VMEM((N_PAD,), jnp.float32),
            pltpu.VMEM((CH,), jnp.int32),
            pltpu.VMEM((CH,), jnp.int32),
            pltpu.VMEM((CH,), jnp.int32),
            pltpu.VMEM((CH,), jnp.int32),
            pltpu.VMEM((B_STATIC, WCH), jnp.float32),
            pltpu.VMEM((B_STATIC, WCH), jnp.float32),
            pltpu.SemaphoreType.DMA,
            pltpu.SemaphoreType.DMA,
            pltpu.SemaphoreType.DMA,
            pltpu.SemaphoreType.DMA,
            pltpu.SemaphoreType.DMA,
            pltpu.SemaphoreType.DMA,
        ],
        compiler_params=pltpu.CompilerParams(needs_layout_passes=False),
    )
    def sc_kernel(eidx_hbm, w_hbm, out_hbm, deg, sb0, sb1, db0, db1,
                  wb0, wb1, ss0, ss1, sd0, sd1, sw0, sw1):
        wid = lax.axis_index("s") * NUM_CORES + lax.axis_index("c")
        b = lax.shift_right_logical(wid, 3)
        g = lax.bitwise_and(wid, 7)
        e_base = g * EDGES_PER_TILE
        src_base = b * (2 * E_STATIC) + e_base
        dst_base = src_base + E_STATIC
        sbufs = (sb0, sb1)
        dbufs = (db0, db1)
        wbufs = (wb0, wb1)
        ssems = (ss0, ss1)
        dsems = (sd0, sd1)
        wsems = (sw0, sw1)
        zeros16 = jnp.zeros((16,), jnp.float32)

        def zero_body(i, carry):
            deg[pl.ds(i * 16, 16)] = zeros16
            return carry
        lax.fori_loop(0, N_PAD // 16, zero_body, 0)

        def fire(c, i):
            off = c * CH
            e0 = e_base + off
            wa = pl.multiple_of(lax.bitwise_and(e0, ~127), 128)
            pltpu.async_copy(eidx_hbm.at[pl.ds(src_base + off, CH)],
                             sbufs[i], ssems[i])
            pltpu.async_copy(eidx_hbm.at[pl.ds(dst_base + off, CH)],
                             dbufs[i], dsems[i])
            pltpu.async_copy(w_hbm.at[:, pl.ds(wa, WCH)],
                             wbufs[i], wsems[i])

        def wait(i):
            pltpu.make_async_copy(eidx_hbm.at[pl.ds(0, CH)],
                                  sbufs[i], ssems[i]).wait()
            pltpu.make_async_copy(eidx_hbm.at[pl.ds(0, CH)],
                                  dbufs[i], dsems[i]).wait()
            pltpu.make_async_copy(w_hbm.at[:, pl.ds(0, WCH)],
                                  wbufs[i], wsems[i]).wait()

        def process(c, i):
            sb = sbufs[i]
            db = dbufs[i]
            wb = wbufs[i]
            woff = lax.bitwise_and(e_base + c * CH, 127)

            def grp_body(q, carry):
                o = q * 80
                for u in range(5):
                    ou = o + u * 16
                    sv = sb[pl.ds(ou, 16)]
                    dv = db[pl.ds(ou, 16)]
                    wv = wb[b, pl.ds(woff + ou, 16)]
                    plsc.addupdate_scatter(deg, [sv], wv)
                    plsc.addupdate_scatter(deg, [dv], wv)
                return carry
            lax.fori_loop(0, CH // 80, grp_body, 0)

        fire(0, 0)

        def pipe_body(t, carry):
            c0 = 2 * t
            c1 = 2 * t + 1
            fire(c1, 1)
            wait(0)
            process(c0, 0)

            @pl.when(c0 + 2 < NCHUNK)
            def _():
                fire(c0 + 2, 0)

            wait(1)
            process(c1, 1)
            return carry
        lax.fori_loop(0, NCHUNK // 2, pipe_body, 0)

        pltpu.sync_copy(deg, out_hbm.at[pl.ds(wid * N_PAD, N_PAD)])

    return sc_kernel(eidx_lin, weights)


def _tc_threshold(partials):
    def body(p_ref, o_ref):
        deg = p_ref[:, 0, :]
        for w in range(1, GROUPS_PER_BATCH):
            deg = deg + p_ref[:, w, :]
        rmax = jnp.max(deg, axis=1, keepdims=True)
        xu = lax.bitcast_convert_type(deg, jnp.uint32)
        col = lax.broadcasted_iota(jnp.int32, (B_STATIC, N_PAD), 1)
        valid = col < N_NODES

        def radix_body(i, p):
            bit = (31 - i).astype(jnp.uint32)
            t = p | (jnp.uint32(1) << bit)
            hit = jnp.logical_and(xu < t, valid)
            cnt = jnp.sum(hit.astype(jnp.int32), axis=1, keepdims=True)
            return jnp.where(cnt <= K_MEDIAN, t, p)

        p = lax.fori_loop(0, 32, radix_body,
                          jnp.zeros((B_STATIC, 1), jnp.uint32))
        kth = lax.bitcast_convert_type(p, jnp.float32)
        denom = rmax + 1e-8
        norm = deg / denom
        thr = kth / denom
        phi = jnp.where(norm >= thr, jnp.float32(1.0), jnp.float32(VARPHI))
        o_ref[...] = phi[:, :N_NODES]

    return pl.pallas_call(
        body,
        out_shape=jax.ShapeDtypeStruct((B_STATIC, N_NODES), jnp.float32),
    )(partials)


def kernel(edge_index, edge_weights, num_nodes):
    eidx_lin = edge_index.transpose(0, 2, 1).reshape(-1)
    partials = _sc_degree_partials(eidx_lin, edge_weights)
    partials = partials.reshape(B_STATIC, GROUPS_PER_BATCH, N_PAD)
    return _tc_threshold(partials)
